# Initial kernel scaffold; baseline (speedup 1.0000x reference)
import jax
import jax.numpy as jnp
from jax.experimental import pallas as pl


def kernel(visual, lookup):
    raise NotImplementedError("write your pallas kernel here")


# speedup vs baseline: 13.5509x; 13.5509x over previous
import functools

import jax
import jax.numpy as jnp
from jax import lax
from jax.experimental import pallas as pl


def _unpack_kernel(x_ref, o_ref):
    x = x_ref[...]
    bB, D = x.shape
    xr = jnp.repeat(x, 8, axis=1)
    shifts = 7 - (lax.broadcasted_iota(jnp.int32, (1, 8 * D), 1) & 7)
    bits = (xr >> shifts) & 1
    o_ref[...] = bits.astype(jnp.float32)


@jax.jit
def kernel(visual, lookup):
    del lookup
    B, D = visual.shape
    block_B = 256
    out = pl.pallas_call(
        _unpack_kernel,
        grid=(B // block_B,),
        in_specs=[pl.BlockSpec((block_B, D), lambda i: (i, 0))],
        out_specs=pl.BlockSpec((block_B, 8 * D), lambda i: (i, 0)),
        out_shape=jax.ShapeDtypeStruct((B, 8 * D), jnp.float32),
    )(visual)
    return out


# speedup vs baseline: 228.7794x; 16.8830x over previous
import functools

import numpy as np
import jax
import jax.numpy as jnp
from jax.experimental import pallas as pl


def _expansion_matrix():
    r = np.zeros((256, 2048), np.float32)
    d = np.arange(256)
    for k in range(8):
        r[d, 8 * d + k] = 2.0 ** (k - 7)
    return jnp.asarray(r, dtype=jnp.bfloat16)


def _unpack_kernel(x_ref, r_ref, o_ref):
    x = x_ref[...].astype(jnp.bfloat16)
    y = jnp.dot(x, r_ref[...], preferred_element_type=jnp.float32)
    o_ref[...] = (y.astype(jnp.int32) & 1).astype(jnp.float32)


@jax.jit
def kernel(visual, lookup):
    del lookup
    B, D = visual.shape
    block_B = 256
    out = pl.pallas_call(
        _unpack_kernel,
        grid=(B // block_B,),
        in_specs=[
            pl.BlockSpec((block_B, D), lambda i: (i, 0)),
            pl.BlockSpec((D, 8 * D), lambda i: (0, 0)),
        ],
        out_specs=pl.BlockSpec((block_B, 8 * D), lambda i: (i, 0)),
        out_shape=jax.ShapeDtypeStruct((B, 8 * D), jnp.float32),
    )(visual, _expansion_matrix())
    return out


# speedup vs baseline: 302.9235x; 1.3241x over previous
import functools

import numpy as np
import jax
import jax.numpy as jnp
from jax.experimental import pallas as pl


def _expansion_matrix():
    r = np.zeros((256, 2048), np.float32)
    d = np.arange(256)
    for k in range(8):
        r[d, 8 * d + k] = 2.0 ** (k - 7)
    return jnp.asarray(r, dtype=jnp.bfloat16)


def _unpack_kernel(x_ref, r_ref, o_ref):
    x = x_ref[...].astype(jnp.bfloat16)
    y = jnp.dot(x, r_ref[...], preferred_element_type=jnp.float32)
    o_ref[...] = (y.astype(jnp.int32) & 1).astype(jnp.float32)


@jax.jit
def kernel(visual, lookup):
    del lookup
    B, D = visual.shape
    block_B = 512
    out = pl.pallas_call(
        _unpack_kernel,
        grid=(B // block_B,),
        in_specs=[
            pl.BlockSpec((block_B, D), lambda i: (i, 0)),
            pl.BlockSpec((D, 8 * D), lambda i: (0, 0)),
        ],
        out_specs=pl.BlockSpec((block_B, 8 * D), lambda i: (i, 0)),
        out_shape=jax.ShapeDtypeStruct((B, 8 * D), jnp.float32),
    )(visual, _expansion_matrix())
    return out


# speedup vs baseline: 345.5696x; 1.1408x over previous
import functools

import numpy as np
import jax
import jax.numpy as jnp
from jax.experimental import pallas as pl


def _expansion_matrix():
    r = np.zeros((256, 2048), np.float32)
    d = np.arange(256)
    for k in range(8):
        r[d, 8 * d + k] = 2.0 ** (k - 7)
    return jnp.asarray(r, dtype=jnp.bfloat16)


def _unpack_kernel(x_ref, r_ref, o_ref):
    x = x_ref[...].astype(jnp.bfloat16)
    y = jnp.dot(x, r_ref[...], preferred_element_type=jnp.float32)
    o_ref[...] = (y.astype(jnp.int32) & 1).astype(jnp.float32)


@jax.jit
def kernel(visual, lookup):
    del lookup
    B, D = visual.shape
    block_B = 1024
    out = pl.pallas_call(
        _unpack_kernel,
        grid=(B // block_B,),
        in_specs=[
            pl.BlockSpec((block_B, D), lambda i: (i, 0)),
            pl.BlockSpec((D, 8 * D), lambda i: (0, 0)),
        ],
        out_specs=pl.BlockSpec((block_B, 8 * D), lambda i: (i, 0)),
        out_shape=jax.ShapeDtypeStruct((B, 8 * D), jnp.float32),
    )(visual, _expansion_matrix())
    return out


# speedup vs baseline: 347.9205x; 1.0068x over previous
import functools

import numpy as np
import jax
import jax.numpy as jnp
from jax.experimental import pallas as pl


def _expansion_matrix():
    r = np.zeros((256, 2048), np.float32)
    d = np.arange(256)
    for k in range(8):
        r[d, 8 * d + k] = 2.0 ** (k - 7)
    return jnp.asarray(r, dtype=jnp.bfloat16)


def _unpack_kernel(x_ref, r_ref, o_ref):
    x = x_ref[...].astype(jnp.bfloat16)
    y = jnp.dot(x, r_ref[...], preferred_element_type=jnp.float32)
    o_ref[...] = (y.astype(jnp.int32) & 1).astype(jnp.float32)


@jax.jit
def kernel(visual, lookup):
    del lookup
    B, D = visual.shape
    block_B = 2048
    out = pl.pallas_call(
        _unpack_kernel,
        grid=(B // block_B,),
        in_specs=[
            pl.BlockSpec((block_B, D), lambda i: (i, 0)),
            pl.BlockSpec((D, 8 * D), lambda i: (0, 0)),
        ],
        out_specs=pl.BlockSpec((block_B, 8 * D), lambda i: (i, 0)),
        out_shape=jax.ShapeDtypeStruct((B, 8 * D), jnp.float32),
    )(visual, _expansion_matrix())
    return out
